# baseline (speedup 1.0000x reference)
import functools

import jax
import jax.numpy as jnp
from jax import lax
from jax.experimental import pallas as pl
from jax.experimental.pallas import tpu as pltpu
from jax.experimental.pallas import tpu_sc as plsc

_NC = 2
_NS = 16
_NW = _NC * _NS

_INT_MAX = 2**31 - 1


def _argmax_body(nb, f_ref, g_ref, out_ref, vscr, iscr):
    j = pl.program_id(0)
    s = f_ref[...] + g_ref[...]
    p, vb = s.shape
    bm = jnp.max(s, axis=1, keepdims=True)
    gidx = j * vb + lax.broadcasted_iota(jnp.int32, (p, vb), 1)
    bi = jnp.min(jnp.where(s == bm, gidx, _INT_MAX), axis=1, keepdims=True)
    bmb = jnp.broadcast_to(bm, (p, 128))
    bib = jnp.broadcast_to(bi, (p, 128))

    @pl.when(j == 0)
    def _():
        vscr[...] = bmb
        iscr[...] = bib

    @pl.when(j > 0)
    def _():
        cur = vscr[...]
        better = bmb > cur
        vscr[...] = jnp.where(better, bmb, cur)
        iscr[...] = jnp.where(better, bib, iscr[...])

    @pl.when(j == nb - 1)
    def _():
        out_ref[...] = iscr[...]


def _argmax_rows(fair, gumbels, vb=65536):
    p, v = fair.shape
    nb = v // vb
    out = pl.pallas_call(
        functools.partial(_argmax_body, nb),
        grid=(nb,),
        in_specs=[
            pl.BlockSpec((p, vb), lambda j: (0, j)),
            pl.BlockSpec((p, vb), lambda j: (0, j)),
        ],
        out_specs=pl.BlockSpec((p, 128), lambda j: (0, 0)),
        out_shape=jax.ShapeDtypeStruct((p, 128), jnp.int32),
        scratch_shapes=[
            pltpu.VMEM((p, 128), jnp.float32),
            pltpu.VMEM((p, 128), jnp.int32),
        ],
    )(fair, gumbels)
    return out[:, 0]


def _sc_gather_body(nch, ids_hbm, table_hbm, out_hbm, idx_v, rows_a, sem_a):
    wid = lax.axis_index("s") * _NC + lax.axis_index("c")
    base = wid * nch

    def body(j, _):
        pltpu.sync_copy(ids_hbm.at[base + j], idx_v)
        pltpu.async_copy(table_hbm.at[idx_v], rows_a, sem_a).wait()
        pltpu.sync_copy(rows_a, out_hbm.at[base + j])
        return _

    lax.fori_loop(0, nch, body, None)


def _sc_gather(ids, table):
    nw, nch, c = ids.shape
    v, d = table.shape
    ids2 = ids.reshape(nw * nch, c)
    mesh = plsc.VectorSubcoreMesh(core_axis_name="c", subcore_axis_name="s")
    k = functools.partial(
        pl.kernel,
        out_type=jax.ShapeDtypeStruct((nw * nch, c, d), jnp.float32),
        mesh=mesh,
        compiler_params=pltpu.CompilerParams(use_tc_tiling_on_sc=False),
        scratch_types=[
            pltpu.VMEM((c,), jnp.int32),
            pltpu.VMEM((c, d), jnp.float32),
            pltpu.SemaphoreType.DMA,
        ],
    )(functools.partial(_sc_gather_body, nch))
    return k(ids2, table)


def kernel(fair_sent_dist, table, gumbels, input_ids, attn_mask):
    p, v = fair_sent_dist.shape
    b, l = input_ids.shape
    d = table.shape[1]

    fair_ids = _argmax_rows(fair_sent_dist, gumbels)

    n = b * (p + l)
    chunk = 112
    assert n % (_NW * chunk) == 0
    nch = n // (_NW * chunk)
    ids_full = jnp.concatenate(
        [jnp.broadcast_to(fair_ids[None, :], (b, p)), input_ids], axis=1
    ).astype(jnp.int32).reshape(_NW, nch, chunk)

    gathered = _sc_gather(ids_full, table)
    out_emb = gathered.reshape(b, p + l, d)

    attention_mask = jnp.concatenate(
        [jnp.ones((b, p), dtype=attn_mask.dtype), attn_mask], axis=1
    )
    return (out_emb, attention_mask)


# speedup vs baseline: 1.2670x; 1.2670x over previous
import functools

import jax
import jax.numpy as jnp
from jax import lax
from jax.experimental import pallas as pl
from jax.experimental.pallas import tpu as pltpu
from jax.experimental.pallas import tpu_sc as plsc

_NC = 2
_NS = 16
_NW = _NC * _NS

_INT_MAX = 2**31 - 1


def _argmax_body(v, f_ref, g_ref, out_ref, vscr):
    j = pl.program_id(0)
    s = f_ref[...] + g_ref[...]
    p, vb = s.shape
    gidx = j * vb + lax.broadcasted_iota(jnp.int32, (p, vb), 1)
    s = jnp.where(gidx < v, s, -jnp.inf)
    bm = jnp.max(s, axis=1, keepdims=True)
    bi = jnp.min(jnp.where(s == bm, gidx, _INT_MAX), axis=1, keepdims=True)
    bmb = jnp.broadcast_to(bm, (p, 128))
    bib = jnp.broadcast_to(bi, (p, 128))

    @pl.when(j == 0)
    def _():
        vscr[...] = bmb
        out_ref[...] = bib

    @pl.when(j > 0)
    def _():
        cur = vscr[...]
        better = bmb > cur
        vscr[...] = jnp.where(better, bmb, cur)
        out_ref[...] = jnp.where(better, bib, out_ref[...])


def _argmax_rows(fair, gumbels, vb=65536):
    p, v = fair.shape
    nb = pl.cdiv(v, vb)
    out = pl.pallas_call(
        functools.partial(_argmax_body, v),
        grid=(nb,),
        in_specs=[
            pl.BlockSpec((p, vb), lambda j: (0, j)),
            pl.BlockSpec((p, vb), lambda j: (0, j)),
        ],
        out_specs=pl.BlockSpec((p, 128), lambda j: (0, 0)),
        out_shape=jax.ShapeDtypeStruct((p, 128), jnp.int32),
        scratch_shapes=[
            pltpu.VMEM((p, 128), jnp.float32),
        ],
    )(fair, gumbels)
    return out[:, 0]


def _sc_gather_body(nch, cr, ids_hbm, table_hbm, out_hbm, idx_v, buf_a, buf_b,
                    sem_ga, sem_gb):
    wid = lax.axis_index("s") * _NC + lax.axis_index("c")
    nrows = nch * cr
    base = wid * nrows
    pltpu.sync_copy(ids_hbm.at[pl.ds(base, nrows)], idx_v)

    def gather_chunk(j, buf, sem):
        def grp(g, _):
            v16 = idx_v[pl.ds(j * cr + 16 * g, 16)]
            for lane in range(16):
                pltpu.make_async_copy(
                    table_hbm.at[v16[lane]], buf.at[16 * g + lane], sem
                ).start()
            return _
        lax.fori_loop(0, cr // 16, grp, None)

    def drain(buf, sem):
        pltpu.make_async_copy(table_hbm.at[pl.ds(0, cr)], buf, sem).wait()

    gather_chunk(0, buf_a, sem_ga)

    def body(k, _):
        j0 = 2 * k
        gather_chunk(j0 + 1, buf_b, sem_gb)
        drain(buf_a, sem_ga)
        pltpu.sync_copy(buf_a, out_hbm.at[pl.ds(base + j0 * cr, cr)])

        @pl.when(j0 + 2 < nch)
        def _():
            gather_chunk(j0 + 2, buf_a, sem_ga)

        drain(buf_b, sem_gb)
        pltpu.sync_copy(buf_b, out_hbm.at[pl.ds(base + (j0 + 1) * cr, cr)])
        return _

    lax.fori_loop(0, nch // 2, body, None)


def _sc_gather(ids, table, cr=240):
    (n,) = ids.shape
    v, d = table.shape
    assert n % (_NW * cr) == 0
    nch = n // (_NW * cr)
    assert nch % 2 == 0 and cr % 16 == 0
    mesh = plsc.VectorSubcoreMesh(core_axis_name="c", subcore_axis_name="s")
    k = functools.partial(
        pl.kernel,
        out_type=jax.ShapeDtypeStruct((n, d), jnp.float32),
        mesh=mesh,
        scratch_types=[
            pltpu.VMEM((nch * cr,), jnp.int32),
            pltpu.VMEM((cr, d), jnp.float32),
            pltpu.VMEM((cr, d), jnp.float32),
            pltpu.SemaphoreType.DMA,
            pltpu.SemaphoreType.DMA,
        ],
    )(functools.partial(_sc_gather_body, nch, cr))
    return k(ids, table)


def kernel(fair_sent_dist, table, gumbels, input_ids, attn_mask):
    p, v = fair_sent_dist.shape
    b, l = input_ids.shape
    d = table.shape[1]

    fair_ids = _argmax_rows(fair_sent_dist, gumbels)

    ids_full = jnp.concatenate(
        [jnp.broadcast_to(fair_ids[None, :], (b, p)), input_ids], axis=1
    ).astype(jnp.int32).reshape(-1)

    out_emb = _sc_gather(ids_full, table).reshape(b, p + l, d)

    attention_mask = jnp.concatenate(
        [jnp.ones((b, p), dtype=attn_mask.dtype), attn_mask], axis=1
    )
    return (out_emb, attention_mask)
